# baseline (speedup 1.0000x reference)
import functools

import jax
import jax.numpy as jnp
from jax import lax
from jax.experimental import pallas as pl
from jax.experimental.pallas import tpu as pltpu
from jax.experimental.pallas import tpu_sc as plsc

L = 16
NROWS = 128
NCOLS = 32768
NW = 32
TC_ROWS = 64
SC_ROWS = NROWS - TC_ROWS
RPW = SC_ROWS // NW
U = 8
GL = U * L
NCHAIN = 8
BIG = 2**30

_GATHER_DNUMS = lax.GatherDimensionNumbers(
    offset_dims=(), collapsed_slice_dims=(0,), start_index_map=(0,))


def _permute(x, perm):
    return lax.gather(x, perm[:, None], _GATHER_DNUMS, (1,),
                      mode=lax.GatherScatterMode.PROMISE_IN_BOUNDS)


def _all_reduce(x, lane, op):
    for s in (1, 2, 4, 8):
        x = op(x, _permute(x, lane ^ s))
    return x


def _finalize_row(m1, i1, m2, i2, lane):
    v1 = _all_reduce(m1, lane, jnp.maximum)
    i1g = _all_reduce(jnp.where(m1 == v1, i1, BIG), lane, jnp.minimum)
    winner = (m1 == v1) & (i1 == i1g)
    c = jnp.where(winner, m2, m1)
    ci = jnp.where(winner, i2, i1)
    v2 = _all_reduce(c, lane, jnp.maximum)
    i2g = _all_reduce(jnp.where(c == v2, ci, BIG), lane, jnp.minimum)
    return v1, i1g, v2, i2g


def _make_sc_kernel():
    mesh = plsc.VectorSubcoreMesh(core_axis_name="c", subcore_axis_name="s")

    @functools.partial(
        pl.kernel,
        out_type=(
            jax.ShapeDtypeStruct((NW * 8,), jnp.float32),
            jax.ShapeDtypeStruct((NW * 8,), jnp.int32),
        ),
        mesh=mesh,
        compiler_params=pltpu.CompilerParams(needs_layout_passes=False),
        scratch_types=[
            pltpu.VMEM((NCOLS,), jnp.float32),
            pltpu.VMEM((NCOLS,), jnp.float32),
            pltpu.VMEM((L,), jnp.float32),
            pltpu.VMEM((L,), jnp.int32),
            pltpu.SemaphoreType.DMA,
            pltpu.SemaphoreType.DMA,
        ],
    )
    def topk2(var_hbm, outv_hbm, outi_hbm, buf0, buf1, resv_ref, resi_ref,
              sem0, sem1):
        wid = lax.axis_index("c") * 16 + lax.axis_index("s")
        base_row = TC_ROWS + wid * RPW
        lane = lax.broadcasted_iota(jnp.int32, (L,), 0)
        neg = jnp.full((L,), -jnp.inf, jnp.float32)
        zero_i = jnp.zeros((L,), jnp.int32)

        bufs = [buf0, buf1]
        sems = [sem0, sem1]
        cps = [None, None]
        cps[0] = pltpu.async_copy(var_hbm.at[base_row], buf0, sem0)

        resv = jnp.zeros((L,), jnp.float32)
        resi = jnp.zeros((L,), jnp.int32)
        for r in range(RPW):
            b = r % 2
            if r + 1 < RPW:
                nb = (r + 1) % 2
                cps[nb] = pltpu.async_copy(
                    var_hbm.at[base_row + r + 1], bufs[nb], sems[nb])
            cps[b].wait()
            buf = bufs[b]

            def update(carry, x, sx):
                m1, i1, m2, i2 = carry
                gt1 = x > m1
                cand = jnp.where(gt1, m1, x)
                candi = jnp.where(gt1, i1, sx)
                m1 = jnp.where(gt1, x, m1)
                i1 = jnp.where(gt1, sx, i1)
                gt2 = (cand > m2) | ((cand == m2) & (candi < i2))
                m2 = jnp.where(gt2, cand, m2)
                i2 = jnp.where(gt2, candi, i2)
                return m1, i1, m2, i2

            def merge_update(carry, x, sx):
                m1, i1, m2, i2 = carry
                gt1 = (x > m1) | ((x == m1) & (sx < i1))
                cand = jnp.where(gt1, m1, x)
                candi = jnp.where(gt1, i1, sx)
                m1 = jnp.where(gt1, x, m1)
                i1 = jnp.where(gt1, sx, i1)
                gt2 = (cand > m2) | ((cand == m2) & (candi < i2))
                m2 = jnp.where(gt2, cand, m2)
                i2 = jnp.where(gt2, candi, i2)
                return m1, i1, m2, i2

            def group_body(g, carry):
                recs = [list(carry[4 * c:4 * c + 4]) for c in range(NCHAIN)]
                base = g * GL
                sb = jnp.full((L,), g * U, jnp.int32)
                for u in range(U):
                    x = buf[pl.ds(base + u * L, L)]
                    c = u % NCHAIN
                    recs[c] = update(recs[c], x, sb + u)
                return tuple(sum([list(rec) for rec in recs], []))

            init = tuple([neg, zero_i, neg, zero_i] * NCHAIN)
            out = plsc.parallel_loop(0, NCOLS // GL, carry=init)(group_body)
            recs = [out[4 * c:4 * c + 4] for c in range(NCHAIN)]
            rec = tuple(recs[0])
            for c in range(1, NCHAIN):
                m1c, i1c, m2c, i2c = recs[c]
                rec = merge_update(rec, m1c, i1c)
                rec = merge_update(rec, m2c, i2c)
            m1, i1, m2, i2 = rec
            i1 = i1 * L + lane
            i2 = i2 * L + lane
            v1, i1g, v2, i2g = _finalize_row(m1, i1, m2, i2, lane)
            resv = jnp.where(lane == 2 * r, v1, resv)
            resv = jnp.where(lane == 2 * r + 1, v2, resv)
            resi = jnp.where(lane == 2 * r, i1g, resi)
            resi = jnp.where(lane == 2 * r + 1, i2g, resi)

        resv_ref[...] = resv
        resi_ref[...] = resi
        pltpu.sync_copy(resv_ref.at[pl.ds(0, 8)],
                        outv_hbm.at[pl.ds(wid * 8, 8)])
        pltpu.sync_copy(resi_ref.at[pl.ds(0, 8)],
                        outi_hbm.at[pl.ds(wid * 8, 8)])

    return topk2


def _tc_topk2_body(x_ref, outv_ref, outi_ref):
    x = x_ref[...]
    iota = lax.broadcasted_iota(jnp.int32, (8, NCOLS), 1)
    m1 = jnp.max(x, axis=1, keepdims=True)
    i1 = jnp.min(jnp.where(x == m1, iota, BIG), axis=1, keepdims=True)
    x2 = jnp.where(iota == i1, -jnp.inf, x)
    m2 = jnp.max(x2, axis=1, keepdims=True)
    i2 = jnp.min(jnp.where(x2 == m2, iota, BIG), axis=1, keepdims=True)
    outv_ref[...] = jnp.concatenate([m1, m2], axis=1)
    outi_ref[...] = jnp.concatenate([i1, i2], axis=1)


def _make_tc_kernel():
    return pl.pallas_call(
        _tc_topk2_body,
        grid=(TC_ROWS // 8,),
        in_specs=[pl.BlockSpec((8, NCOLS), lambda i: (i, 0))],
        out_specs=[pl.BlockSpec((8, 2), lambda i: (i, 0)),
                   pl.BlockSpec((8, 2), lambda i: (i, 0))],
        out_shape=[jax.ShapeDtypeStruct((TC_ROWS, 2), jnp.float32),
                   jax.ShapeDtypeStruct((TC_ROWS, 2), jnp.int32)],
    )


_topk2_sc = _make_sc_kernel()
_topk2_tc = _make_tc_kernel()


@jax.jit
def kernel(var):
    sc_v, sc_i = _topk2_sc(var)
    tc_v, tc_i = _topk2_tc(var)
    sc_v = sc_v.reshape(NW, 8)[:, :2 * RPW].reshape(SC_ROWS, 2)
    sc_i = sc_i.reshape(NW, 8)[:, :2 * RPW].reshape(SC_ROWS, 2)
    v = jnp.concatenate([tc_v, sc_v], axis=0)
    i = jnp.concatenate([tc_i, sc_i], axis=0)
    return v, i


# speedup vs baseline: 1.0732x; 1.0732x over previous
import functools

import jax
import jax.numpy as jnp
from jax import lax
from jax.experimental import pallas as pl
from jax.experimental.pallas import tpu as pltpu
from jax.experimental.pallas import tpu_sc as plsc

L = 16
NROWS = 128
NCOLS = 32768
NW = 32
TC_ROWS = 64
SC_ROWS = NROWS - TC_ROWS
RPW = SC_ROWS // NW
U = 8
GL = U * L
NCHAIN = 4
BIG = 2**30

_GATHER_DNUMS = lax.GatherDimensionNumbers(
    offset_dims=(), collapsed_slice_dims=(0,), start_index_map=(0,))


def _permute(x, perm):
    return lax.gather(x, perm[:, None], _GATHER_DNUMS, (1,),
                      mode=lax.GatherScatterMode.PROMISE_IN_BOUNDS)


def _all_reduce(x, lane, op):
    for s in (1, 2, 4, 8):
        x = op(x, _permute(x, lane ^ s))
    return x


def _finalize_row(m1, i1, m2, i2, lane):
    v1 = _all_reduce(m1, lane, jnp.maximum)
    i1g = _all_reduce(jnp.where(m1 == v1, i1, BIG), lane, jnp.minimum)
    winner = (m1 == v1) & (i1 == i1g)
    c = jnp.where(winner, m2, m1)
    ci = jnp.where(winner, i2, i1)
    v2 = _all_reduce(c, lane, jnp.maximum)
    i2g = _all_reduce(jnp.where(c == v2, ci, BIG), lane, jnp.minimum)
    return v1, i1g, v2, i2g


def _make_sc_kernel():
    mesh = plsc.VectorSubcoreMesh(core_axis_name="c", subcore_axis_name="s")

    @functools.partial(
        pl.kernel,
        out_type=(
            jax.ShapeDtypeStruct((NW * 8,), jnp.float32),
            jax.ShapeDtypeStruct((NW * 8,), jnp.int32),
        ),
        mesh=mesh,
        compiler_params=pltpu.CompilerParams(needs_layout_passes=False),
        scratch_types=[
            pltpu.VMEM((NCOLS,), jnp.float32),
            pltpu.VMEM((NCOLS,), jnp.float32),
            pltpu.VMEM((L,), jnp.float32),
            pltpu.VMEM((L,), jnp.int32),
            pltpu.SemaphoreType.DMA,
            pltpu.SemaphoreType.DMA,
        ],
    )
    def topk2(var_hbm, outv_hbm, outi_hbm, buf0, buf1, resv_ref, resi_ref,
              sem0, sem1):
        wid = lax.axis_index("c") * 16 + lax.axis_index("s")
        base_row = TC_ROWS + wid * RPW
        lane = lax.broadcasted_iota(jnp.int32, (L,), 0)
        neg = jnp.full((L,), -jnp.inf, jnp.float32)
        zero_i = jnp.zeros((L,), jnp.int32)

        bufs = [buf0, buf1]
        sems = [sem0, sem1]
        cps = [None, None]
        cps[0] = pltpu.async_copy(var_hbm.at[base_row], buf0, sem0)

        resv = jnp.zeros((L,), jnp.float32)
        resi = jnp.zeros((L,), jnp.int32)
        for r in range(RPW):
            b = r % 2
            if r + 1 < RPW:
                nb = (r + 1) % 2
                cps[nb] = pltpu.async_copy(
                    var_hbm.at[base_row + r + 1], bufs[nb], sems[nb])
            cps[b].wait()
            buf = bufs[b]

            def update(carry, x, sx):
                m1, i1, m2, i2 = carry
                gt1 = x > m1
                cand = jnp.where(gt1, m1, x)
                candi = jnp.where(gt1, i1, sx)
                m1 = jnp.where(gt1, x, m1)
                i1 = jnp.where(gt1, sx, i1)
                gt2 = (cand > m2) | ((cand == m2) & (candi < i2))
                m2 = jnp.where(gt2, cand, m2)
                i2 = jnp.where(gt2, candi, i2)
                return m1, i1, m2, i2

            def merge_update(carry, x, sx):
                m1, i1, m2, i2 = carry
                gt1 = (x > m1) | ((x == m1) & (sx < i1))
                cand = jnp.where(gt1, m1, x)
                candi = jnp.where(gt1, i1, sx)
                m1 = jnp.where(gt1, x, m1)
                i1 = jnp.where(gt1, sx, i1)
                gt2 = (cand > m2) | ((cand == m2) & (candi < i2))
                m2 = jnp.where(gt2, cand, m2)
                i2 = jnp.where(gt2, candi, i2)
                return m1, i1, m2, i2

            def group_body(g, carry):
                recs = [list(carry[4 * c:4 * c + 4]) for c in range(NCHAIN)]
                base = g * GL
                sb = jnp.full((L,), g * U, jnp.int32)
                for u in range(U):
                    x = buf[pl.ds(base + u * L, L)]
                    c = u % NCHAIN
                    recs[c] = update(recs[c], x, sb + u)
                return tuple(sum([list(rec) for rec in recs], []))

            init = tuple([neg, zero_i, neg, zero_i] * NCHAIN)
            out = plsc.parallel_loop(0, NCOLS // GL, carry=init)(group_body)
            recs = [out[4 * c:4 * c + 4] for c in range(NCHAIN)]
            rec = tuple(recs[0])
            for c in range(1, NCHAIN):
                m1c, i1c, m2c, i2c = recs[c]
                rec = merge_update(rec, m1c, i1c)
                rec = merge_update(rec, m2c, i2c)
            m1, i1, m2, i2 = rec
            i1 = i1 * L + lane
            i2 = i2 * L + lane
            v1, i1g, v2, i2g = _finalize_row(m1, i1, m2, i2, lane)
            resv = jnp.where(lane == 2 * r, v1, resv)
            resv = jnp.where(lane == 2 * r + 1, v2, resv)
            resi = jnp.where(lane == 2 * r, i1g, resi)
            resi = jnp.where(lane == 2 * r + 1, i2g, resi)

        resv_ref[...] = resv
        resi_ref[...] = resi
        pltpu.sync_copy(resv_ref.at[pl.ds(0, 8)],
                        outv_hbm.at[pl.ds(wid * 8, 8)])
        pltpu.sync_copy(resi_ref.at[pl.ds(0, 8)],
                        outi_hbm.at[pl.ds(wid * 8, 8)])

    return topk2


def _tc_topk2_body(x_ref, outv_ref, outi_ref):
    x = x_ref[...]
    iota = lax.broadcasted_iota(jnp.int32, (8, NCOLS), 1)
    m1 = jnp.max(x, axis=1, keepdims=True)
    i1 = jnp.min(jnp.where(x == m1, iota, BIG), axis=1, keepdims=True)
    x2 = jnp.where(iota == i1, -jnp.inf, x)
    m2 = jnp.max(x2, axis=1, keepdims=True)
    i2 = jnp.min(jnp.where(x2 == m2, iota, BIG), axis=1, keepdims=True)
    outv_ref[...] = jnp.concatenate([m1, m2], axis=1)
    outi_ref[...] = jnp.concatenate([i1, i2], axis=1)


def _make_tc_kernel():
    return pl.pallas_call(
        _tc_topk2_body,
        grid=(TC_ROWS // 8,),
        in_specs=[pl.BlockSpec((8, NCOLS), lambda i: (i, 0))],
        out_specs=[pl.BlockSpec((8, 2), lambda i: (i, 0)),
                   pl.BlockSpec((8, 2), lambda i: (i, 0))],
        out_shape=[jax.ShapeDtypeStruct((TC_ROWS, 2), jnp.float32),
                   jax.ShapeDtypeStruct((TC_ROWS, 2), jnp.int32)],
    )


_topk2_sc = _make_sc_kernel()
_topk2_tc = _make_tc_kernel()


@jax.jit
def kernel(var):
    sc_v, sc_i = _topk2_sc(var)
    tc_v, tc_i = _topk2_tc(var)
    sc_v = sc_v.reshape(NW, 8)[:, :2 * RPW].reshape(SC_ROWS, 2)
    sc_i = sc_i.reshape(NW, 8)[:, :2 * RPW].reshape(SC_ROWS, 2)
    v = jnp.concatenate([tc_v, sc_v], axis=0)
    i = jnp.concatenate([tc_i, sc_i], axis=0)
    return v, i
